# Initial kernel scaffold; baseline (speedup 1.0000x reference)
import jax
import jax.numpy as jnp
from jax.experimental import pallas as pl


def kernel(nodes, h, edge_dst, W, b):
    raise NotImplementedError("write your pallas kernel here")


# speedup vs baseline: 3.5699x; 3.5699x over previous
import functools

import jax
import jax.numpy as jnp
from jax import lax
from jax.experimental import pallas as pl
from jax.experimental.pallas import tpu as pltpu
from jax.experimental.pallas import tpu_sc as plsc

_LANES = 128


def _make_segsum(N, E, D):
    DC = D // 2
    NCH = E // _LANES
    NTILES = 16
    base_per_tile = NCH // NTILES
    rem = NCH - base_per_tile * NTILES
    K = 3
    ngroups = base_per_tile // K
    tail = base_per_tile - ngroups * K
    rows_main = (N // NTILES) // 8 * 8
    rows_tail = N - rows_main * NTILES

    mesh = plsc.VectorSubcoreMesh(core_axis_name="c", subcore_axis_name="s")

    @functools.partial(
        pl.kernel,
        mesh=mesh,
        out_type=jax.ShapeDtypeStruct((N, D), jnp.float32),
        scratch_types=[
            pltpu.VMEM((K, _LANES), jnp.int32),
            pltpu.VMEM((K * _LANES, DC), jnp.float32),
            pltpu.VMEM_SHARED((N, DC), jnp.float32),
        ],
    )
    def segsum(h_hbm, dst_hbm, zeros_hbm, out_hbm, idx_v, hbuf, acc_sh):
        cc = lax.axis_index("c")
        sid = lax.axis_index("s")
        col0 = cc * DC
        r0 = sid * rows_main

        pltpu.sync_copy(zeros_hbm, acc_sh.at[pl.ds(r0, rows_main)])

        @pl.when(sid == NTILES - 1)
        def _():
            pltpu.sync_copy(
                zeros_hbm.at[pl.ds(0, rows_tail)],
                acc_sh.at[pl.ds(NTILES * rows_main, rows_tail)],
            )

        plsc.subcore_barrier()

        chunk_base = sid * base_per_tile

        def do_group(ch0, nk):
            e0 = ch0 * _LANES
            for j in range(nk):
                pltpu.sync_copy(
                    dst_hbm.at[pl.ds(e0 + j * _LANES, _LANES)], idx_v.at[j]
                )
            pltpu.sync_copy(
                h_hbm.at[pl.ds(e0, nk * _LANES), pl.ds(col0, DC)],
                hbuf.at[pl.ds(0, nk * _LANES)],
            )
            for j in range(nk):
                pltpu.sync_copy(
                    hbuf.at[pl.ds(j * _LANES, _LANES)],
                    acc_sh.at[idx_v.at[j]],
                    add=True,
                )

        def gbody(g, carry):
            do_group(chunk_base + g * K, K)
            return carry

        lax.fori_loop(0, ngroups, gbody, 0)
        if tail:
            do_group(chunk_base + ngroups * K, tail)

        if rem:
            @pl.when(sid < rem)
            def _():
                do_group(NTILES * base_per_tile + sid, 1)

        plsc.subcore_barrier()
        pltpu.sync_copy(
            acc_sh.at[pl.ds(r0, rows_main)],
            out_hbm.at[pl.ds(r0, rows_main), pl.ds(col0, DC)],
        )

        @pl.when(sid == NTILES - 1)
        def _():
            pltpu.sync_copy(
                acc_sh.at[pl.ds(NTILES * rows_main, rows_tail)],
                out_hbm.at[pl.ds(NTILES * rows_main, rows_tail),
                           pl.ds(col0, DC)],
            )

    return segsum


def _make_linrelu(M, K1, K2, DO):
    BM = 1000

    def body(nodes_ref, si_ref, w1_ref, w2_ref, b_ref, o_ref):
        acc = jnp.dot(nodes_ref[...], w1_ref[...],
                      preferred_element_type=jnp.float32)
        acc = acc + jnp.dot(si_ref[...], w2_ref[...],
                            preferred_element_type=jnp.float32)
        o_ref[...] = jnp.maximum(acc + b_ref[...], 0.0)

    return pl.pallas_call(
        body,
        grid=(M // BM,),
        in_specs=[
            pl.BlockSpec((BM, K1), lambda i: (i, 0)),
            pl.BlockSpec((BM, K2), lambda i: (i, 0)),
            pl.BlockSpec((K1, DO), lambda i: (0, 0)),
            pl.BlockSpec((K2, DO), lambda i: (0, 0)),
            pl.BlockSpec((1, DO), lambda i: (0, 0)),
        ],
        out_specs=pl.BlockSpec((BM, DO), lambda i: (i, 0)),
        out_shape=jax.ShapeDtypeStruct((M, DO), jnp.float32),
    )


def kernel(nodes, h, edge_dst, W, b):
    N, DN = nodes.shape
    E, DE = h.shape
    DO = W.shape[1]

    zeros = jnp.zeros((624, DE // 2), jnp.float32)

    sum_inc = _make_segsum(N, E, DE)(h, edge_dst, zeros)

    out = _make_linrelu(N, DN, DE, DO)(
        nodes, sum_inc, W[:DN], W[DN:], b.reshape(1, DO)
    )
    return out


# speedup vs baseline: 5.2424x; 1.4685x over previous
import functools

import jax
import jax.numpy as jnp
from jax import lax
from jax.experimental import pallas as pl
from jax.experimental.pallas import tpu as pltpu
from jax.experimental.pallas import tpu_sc as plsc

_LANES = 128


def _make_segsum(N, E, D):
    DC = D // 2
    NCH = E // _LANES
    NTILES = 16
    base_per_tile = NCH // NTILES
    rem = NCH - base_per_tile * NTILES
    NBUF = 2
    npairs = base_per_tile // NBUF
    assert npairs * NBUF == base_per_tile
    rows_main = (N // NTILES) // 8 * 8
    rows_tail = N - rows_main * NTILES

    mesh = plsc.VectorSubcoreMesh(core_axis_name="c", subcore_axis_name="s")

    @functools.partial(
        pl.kernel,
        mesh=mesh,
        out_type=jax.ShapeDtypeStruct((N, D), jnp.float32),
        scratch_types=[
            pltpu.VMEM((NBUF, _LANES), jnp.int32),
            pltpu.VMEM((NBUF * _LANES, DC), jnp.float32),
            pltpu.VMEM_SHARED((N, DC), jnp.float32),
            pltpu.SemaphoreType.DMA,
            pltpu.SemaphoreType.DMA,
        ],
    )
    def segsum(h_hbm, dst_hbm, zeros_hbm, out_hbm, idx_v, hbuf, acc_sh,
               sem0, sem1):
        cc = lax.axis_index("c")
        sid = lax.axis_index("s")
        col0 = cc * DC
        r0 = sid * rows_main
        sems = (sem0, sem1)

        pltpu.sync_copy(zeros_hbm, acc_sh.at[pl.ds(r0, rows_main)])

        @pl.when(sid == NTILES - 1)
        def _():
            pltpu.sync_copy(
                zeros_hbm.at[pl.ds(0, rows_tail)],
                acc_sh.at[pl.ds(NTILES * rows_main, rows_tail)],
            )

        plsc.subcore_barrier()

        chunk_base = sid * base_per_tile
        last_chunk = chunk_base + base_per_tile - 1

        def copies(b, ch):
            e0 = ch * _LANES
            return (
                pltpu.make_async_copy(
                    h_hbm.at[pl.ds(e0, _LANES), pl.ds(col0, DC)],
                    hbuf.at[pl.ds(b * _LANES, _LANES)],
                    sems[b],
                ),
                pltpu.make_async_copy(
                    dst_hbm.at[pl.ds(e0, _LANES)], idx_v.at[b], sems[b]
                ),
            )

        def start_load(b, ch):
            for c in copies(b, ch):
                c.start()

        def wait_load(b, ch):
            for c in copies(b, ch):
                c.wait()

        for b in range(NBUF):
            start_load(b, chunk_base + b)

        def pbody(p, carry):
            for b in range(NBUF):
                ch = chunk_base + p * NBUF + b
                wait_load(b, ch)
                pltpu.sync_copy(
                    hbuf.at[pl.ds(b * _LANES, _LANES)],
                    acc_sh.at[idx_v.at[b]],
                    add=True,
                )
                start_load(b, jnp.minimum(ch + NBUF, last_chunk))
            return carry

        lax.fori_loop(0, npairs, pbody, 0)

        for b in range(NBUF):
            wait_load(b, last_chunk)

        if rem:
            @pl.when(sid < rem)
            def _():
                ch = NTILES * base_per_tile + sid
                e0 = ch * _LANES
                pltpu.sync_copy(dst_hbm.at[pl.ds(e0, _LANES)], idx_v.at[0])
                pltpu.sync_copy(
                    h_hbm.at[pl.ds(e0, _LANES), pl.ds(col0, DC)],
                    hbuf.at[pl.ds(0, _LANES)],
                )
                pltpu.sync_copy(
                    hbuf.at[pl.ds(0, _LANES)],
                    acc_sh.at[idx_v.at[0]],
                    add=True,
                )

        plsc.subcore_barrier()
        pltpu.sync_copy(
            acc_sh.at[pl.ds(r0, rows_main)],
            out_hbm.at[pl.ds(r0, rows_main), pl.ds(col0, DC)],
        )

        @pl.when(sid == NTILES - 1)
        def _():
            pltpu.sync_copy(
                acc_sh.at[pl.ds(NTILES * rows_main, rows_tail)],
                out_hbm.at[pl.ds(NTILES * rows_main, rows_tail),
                           pl.ds(col0, DC)],
            )

    return segsum


def _make_linrelu(M, K1, K2, DO):
    BM = 1000

    def body(nodes_ref, si_ref, w1_ref, w2_ref, b_ref, o_ref):
        acc = jnp.dot(nodes_ref[...], w1_ref[...],
                      preferred_element_type=jnp.float32)
        acc = acc + jnp.dot(si_ref[...], w2_ref[...],
                            preferred_element_type=jnp.float32)
        o_ref[...] = jnp.maximum(acc + b_ref[...], 0.0)

    return pl.pallas_call(
        body,
        grid=(M // BM,),
        in_specs=[
            pl.BlockSpec((BM, K1), lambda i: (i, 0)),
            pl.BlockSpec((BM, K2), lambda i: (i, 0)),
            pl.BlockSpec((K1, DO), lambda i: (0, 0)),
            pl.BlockSpec((K2, DO), lambda i: (0, 0)),
            pl.BlockSpec((1, DO), lambda i: (0, 0)),
        ],
        out_specs=pl.BlockSpec((BM, DO), lambda i: (i, 0)),
        out_shape=jax.ShapeDtypeStruct((M, DO), jnp.float32),
    )


def kernel(nodes, h, edge_dst, W, b):
    N, DN = nodes.shape
    E, DE = h.shape
    DO = W.shape[1]

    zeros = jnp.zeros((624, DE // 2), jnp.float32)

    sum_inc = _make_segsum(N, E, DE)(h, edge_dst, zeros)

    out = _make_linrelu(N, DN, DE, DO)(
        nodes, sum_inc, W[:DN], W[DN:], b.reshape(1, DO)
    )
    return out


# speedup vs baseline: 5.4968x; 1.0485x over previous
import functools

import jax
import jax.numpy as jnp
from jax import lax
from jax.experimental import pallas as pl
from jax.experimental.pallas import tpu as pltpu
from jax.experimental.pallas import tpu_sc as plsc

_LANES = 128


def _make_segsum(N, E, D):
    DC = D // 2
    NCH = E // _LANES
    NTILES = 16
    base_per_tile = NCH // NTILES
    rem = NCH - base_per_tile * NTILES
    NBUF = 3
    npairs = base_per_tile // NBUF
    assert npairs * NBUF == base_per_tile
    rows_main = (N // NTILES) // 8 * 8
    rows_tail = N - rows_main * NTILES

    mesh = plsc.VectorSubcoreMesh(core_axis_name="c", subcore_axis_name="s")

    @functools.partial(
        pl.kernel,
        mesh=mesh,
        out_type=jax.ShapeDtypeStruct((N, D), jnp.float32),
        scratch_types=[
            pltpu.VMEM((NBUF, _LANES), jnp.int32),
            pltpu.VMEM((NBUF * _LANES, DC), jnp.float32),
            pltpu.VMEM_SHARED((N, DC), jnp.float32),
            pltpu.SemaphoreType.DMA,
            pltpu.SemaphoreType.DMA,
            pltpu.SemaphoreType.DMA,
        ],
    )
    def segsum(h_hbm, dst_hbm, zeros_hbm, out_hbm, idx_v, hbuf, acc_sh,
               sem0, sem1, sem2):
        cc = lax.axis_index("c")
        sid = lax.axis_index("s")
        col0 = cc * DC
        r0 = sid * rows_main
        sems = (sem0, sem1, sem2)

        pltpu.sync_copy(zeros_hbm, acc_sh.at[pl.ds(r0, rows_main)])

        @pl.when(sid == NTILES - 1)
        def _():
            pltpu.sync_copy(
                zeros_hbm.at[pl.ds(0, rows_tail)],
                acc_sh.at[pl.ds(NTILES * rows_main, rows_tail)],
            )

        plsc.subcore_barrier()

        chunk_base = sid * base_per_tile
        last_chunk = chunk_base + base_per_tile - 1

        def copies(b, ch):
            e0 = ch * _LANES
            return (
                pltpu.make_async_copy(
                    h_hbm.at[pl.ds(e0, _LANES), pl.ds(col0, DC)],
                    hbuf.at[pl.ds(b * _LANES, _LANES)],
                    sems[b],
                ),
                pltpu.make_async_copy(
                    dst_hbm.at[pl.ds(e0, _LANES)], idx_v.at[b], sems[b]
                ),
            )

        def start_load(b, ch):
            for c in copies(b, ch):
                c.start()

        def wait_load(b, ch):
            for c in copies(b, ch):
                c.wait()

        for b in range(NBUF):
            start_load(b, chunk_base + b)

        def pbody(p, carry):
            for b in range(NBUF):
                ch = chunk_base + p * NBUF + b
                wait_load(b, ch)
                pltpu.sync_copy(
                    hbuf.at[pl.ds(b * _LANES, _LANES)],
                    acc_sh.at[idx_v.at[b]],
                    add=True,
                )
                start_load(b, jnp.minimum(ch + NBUF, last_chunk))
            return carry

        lax.fori_loop(0, npairs, pbody, 0)

        for b in range(NBUF):
            wait_load(b, last_chunk)

        if rem:
            @pl.when(sid < rem)
            def _():
                ch = NTILES * base_per_tile + sid
                e0 = ch * _LANES
                pltpu.sync_copy(dst_hbm.at[pl.ds(e0, _LANES)], idx_v.at[0])
                pltpu.sync_copy(
                    h_hbm.at[pl.ds(e0, _LANES), pl.ds(col0, DC)],
                    hbuf.at[pl.ds(0, _LANES)],
                )
                pltpu.sync_copy(
                    hbuf.at[pl.ds(0, _LANES)],
                    acc_sh.at[idx_v.at[0]],
                    add=True,
                )

        plsc.subcore_barrier()
        pltpu.sync_copy(
            acc_sh.at[pl.ds(r0, rows_main)],
            out_hbm.at[pl.ds(r0, rows_main), pl.ds(col0, DC)],
        )

        @pl.when(sid == NTILES - 1)
        def _():
            pltpu.sync_copy(
                acc_sh.at[pl.ds(NTILES * rows_main, rows_tail)],
                out_hbm.at[pl.ds(NTILES * rows_main, rows_tail),
                           pl.ds(col0, DC)],
            )

    return segsum


def _make_linrelu(M, K1, K2, DO):
    BM = 1000

    def body(nodes_ref, si_ref, w1_ref, w2_ref, b_ref, o_ref):
        acc = jnp.dot(nodes_ref[...], w1_ref[...],
                      preferred_element_type=jnp.float32)
        acc = acc + jnp.dot(si_ref[...], w2_ref[...],
                            preferred_element_type=jnp.float32)
        o_ref[...] = jnp.maximum(acc + b_ref[...], 0.0)

    return pl.pallas_call(
        body,
        grid=(M // BM,),
        in_specs=[
            pl.BlockSpec((BM, K1), lambda i: (i, 0)),
            pl.BlockSpec((BM, K2), lambda i: (i, 0)),
            pl.BlockSpec((K1, DO), lambda i: (0, 0)),
            pl.BlockSpec((K2, DO), lambda i: (0, 0)),
            pl.BlockSpec((1, DO), lambda i: (0, 0)),
        ],
        out_specs=pl.BlockSpec((BM, DO), lambda i: (i, 0)),
        out_shape=jax.ShapeDtypeStruct((M, DO), jnp.float32),
    )


def kernel(nodes, h, edge_dst, W, b):
    N, DN = nodes.shape
    E, DE = h.shape
    DO = W.shape[1]

    zeros = jnp.zeros((624, DE // 2), jnp.float32)

    sum_inc = _make_segsum(N, E, DE)(h, edge_dst, zeros)

    out = _make_linrelu(N, DN, DE, DO)(
        nodes, sum_inc, W[:DN], W[DN:], b.reshape(1, DO)
    )
    return out
